# baseline (speedup 1.0000x reference)
import functools

import jax
import jax.numpy as jnp
from jax import lax
from jax.experimental import pallas as pl
from jax.experimental.pallas import tpu as pltpu
from jax.experimental.pallas import tpu_sc as plsc

_N = 10000
_E = 320000
_D = 128
_T = 4
_DT = _D * _T

_NW = 32
_K = 25
_CPB = 40
_BE = _CPB * _K
_NBLK = 10
_EPW = _NBLK * _BE
_ROWS = _E // _K
_NPAD = 10112
_RPT = _NPAD // 16

_BLK = 1000


def _stats_body(x_ref, o_ref):
    @pl.when(pl.program_id(0) == 0)
    def _init():
        o_ref[...] = jnp.zeros_like(o_ref)

    x = x_ref[...]
    o_ref[0:1, :] += jnp.sum(x, axis=0, keepdims=True)
    o_ref[1:2, :] += jnp.sum(x * x, axis=0, keepdims=True)


def _stats(x):
    return pl.pallas_call(
        _stats_body,
        grid=(_N // _BLK,),
        in_specs=[pl.BlockSpec((_BLK, _D), lambda i: (i, 0))],
        out_specs=pl.BlockSpec((2, _D), lambda i: (0, 0)),
        out_shape=jax.ShapeDtypeStruct((2, _D), jnp.float32),
    )(x)


def _sum_stats_body(p0_ref, p1_ref, h_ref, o_ref):
    @pl.when(pl.program_id(0) == 0)
    def _init():
        o_ref[...] = jnp.zeros_like(o_ref)

    h = p0_ref[0] + p1_ref[0]
    h_ref[...] = h
    o_ref[0:1, :] += jnp.sum(h, axis=0, keepdims=True)
    o_ref[1:2, :] += jnp.sum(h * h, axis=0, keepdims=True)


def _sum_stats(parts):
    return pl.pallas_call(
        _sum_stats_body,
        grid=(_N // _BLK,),
        in_specs=[
            pl.BlockSpec((1, _BLK, _D), lambda i: (0, i, 0)),
            pl.BlockSpec((1, _BLK, _D), lambda i: (1, i, 0)),
        ],
        out_specs=[
            pl.BlockSpec((_BLK, _D), lambda i: (i, 0)),
            pl.BlockSpec((2, _D), lambda i: (0, 0)),
        ],
        out_shape=[
            jax.ShapeDtypeStruct((_N, _D), jnp.float32),
            jax.ShapeDtypeStruct((2, _D), jnp.float32),
        ],
    )(parts, parts)


def _bn_mm_body(st_ref, gb_ref, b_ref, x_ref, w_ref, o_ref):
    inv_n = 1.0 / _N
    mean = st_ref[0:1, :] * inv_n
    var = st_ref[1:2, :] * inv_n - mean * mean
    scale = lax.rsqrt(var + 1e-5) * gb_ref[0:1, :]
    shift = gb_ref[1:2, :] - mean * scale
    xr = jnp.maximum(x_ref[...] * scale + shift, 0.0)
    o_ref[...] = (
        jnp.dot(xr, w_ref[...], preferred_element_type=jnp.float32) + b_ref[...]
    )


def _bn_mm(st, gb, b, x, w):
    return pl.pallas_call(
        _bn_mm_body,
        grid=(_N // _BLK,),
        in_specs=[
            pl.BlockSpec((2, _D), lambda i: (0, 0)),
            pl.BlockSpec((2, _D), lambda i: (0, 0)),
            pl.BlockSpec((1, _DT), lambda i: (0, 0)),
            pl.BlockSpec((_BLK, _D), lambda i: (i, 0)),
            pl.BlockSpec((_D, _DT), lambda i: (0, 0)),
        ],
        out_specs=pl.BlockSpec((_BLK, _DT), lambda i: (i, 0)),
        out_shape=jax.ShapeDtypeStruct((_N, _DT), jnp.float32),
    )(st, gb, b, x, w)


def _et_prep_body(x_ref, o_ref):
    o_ref[...] = x_ref[...]


def _et_prep(et):
    return pl.pallas_call(
        _et_prep_body,
        grid=(160,),
        in_specs=[pl.BlockSpec((_E // 160, _T), lambda i: (i, 0))],
        out_specs=pl.BlockSpec((_E // 160, _T), lambda i: (i, 0)),
        out_shape=jax.ShapeDtypeStruct((_E, _T), jnp.float32),
    )(et)


def _final_body(p0_ref, p1_ref, f_ref, o_ref):
    o_ref[...] = p0_ref[0] + p1_ref[0] + f_ref[...]


def _final(parts, f):
    return pl.pallas_call(
        _final_body,
        grid=(_N // _BLK,),
        in_specs=[
            pl.BlockSpec((1, _BLK, _D), lambda i: (0, i, 0)),
            pl.BlockSpec((1, _BLK, _D), lambda i: (1, i, 0)),
            pl.BlockSpec((_BLK, _D), lambda i: (i, 0)),
        ],
        out_specs=pl.BlockSpec((_BLK, _D), lambda i: (i, 0)),
        out_shape=jax.ShapeDtypeStruct((_BLK * (_N // _BLK), _D), jnp.float32),
    )(parts, parts, f)


_mesh = plsc.VectorSubcoreMesh(core_axis_name="c", subcore_axis_name="s")


@functools.partial(
    pl.kernel,
    out_type=jax.ShapeDtypeStruct((2, _NPAD, _D), jnp.float32),
    mesh=_mesh,
    scratch_types=[
        pltpu.VMEM_SHARED((_NPAD, _D), jnp.float32),
        pltpu.VMEM((_CPB, _K), jnp.int32),
        pltpu.VMEM((_CPB, _K), jnp.int32),
        pltpu.VMEM((_BE, _T), jnp.float32),
        pltpu.VMEM((_K, _DT), jnp.float32),
        pltpu.VMEM((_K, _DT), jnp.float32),
        pltpu.VMEM((_K, _D), jnp.float32),
        pltpu.VMEM((_K, _D), jnp.float32),
        pltpu.SemaphoreType.DMA,
        pltpu.SemaphoreType.DMA,
        pltpu.SemaphoreType.DMA,
        pltpu.SemaphoreType.DMA,
    ],
    compiler_params=pltpu.CompilerParams(
        use_tc_tiling_on_sc=False, needs_layout_passes=False
    ),
)
def _propagate(h_hbm, src_hbm, dst_hbm, et_hbm, out_hbm,
               acc, src_v, dst_v, et_v, rows_a, rows_b, msg_a, msg_b,
               sem_ga, sem_gb, sem_sa, sem_sb):
    c = lax.axis_index("c")
    s = lax.axis_index("s")
    wid = s * 2 + c

    def _zb(i, _):
        msg_a[i // 8, pl.ds((i % 8) * 16, 16)] = jnp.zeros((16,), jnp.float32)
        return 0

    lax.fori_loop(0, _K * 8, _zb, 0)

    def _zc(j, _):
        pltpu.sync_copy(msg_a, acc.at[pl.ds(s * _RPT + j * _K, _K)])
        return 0

    lax.fori_loop(0, _RPT // _K, _zc, 0)
    pltpu.sync_copy(msg_a.at[pl.ds(0, _RPT % _K)],
                    acc.at[pl.ds(s * _RPT + (_RPT // _K) * _K, _RPT % _K)])

    plsc.subcore_barrier()

    def _compute(ci, rows, msg):
        @plsc.parallel_loop(0, _K, 1, unroll=2)
        def _edge(k):
            e_i = jnp.full((16,), ci * _K + k, jnp.int32)
            w0 = plsc.load_gather(et_v, [e_i, jnp.full((16,), 0, jnp.int32)])
            w1 = plsc.load_gather(et_v, [e_i, jnp.full((16,), 1, jnp.int32)])
            w2 = plsc.load_gather(et_v, [e_i, jnp.full((16,), 2, jnp.int32)])
            w3 = plsc.load_gather(et_v, [e_i, jnp.full((16,), 3, jnp.int32)])
            for g in range(_D // 16):
                o = g * 16
                m = w0 * rows[k, pl.ds(o, 16)]
                m = m + w1 * rows[k, pl.ds(_D + o, 16)]
                m = m + w2 * rows[k, pl.ds(2 * _D + o, 16)]
                m = m + w3 * rows[k, pl.ds(3 * _D + o, 16)]
                msg[k, pl.ds(o, 16)] = m

    def _block(b, _):
        base = wid * _EPW + b * _BE
        rbase = wid * (_EPW // _K) + b * _CPB

        pltpu.sync_copy(src_hbm.at[pl.ds(rbase, _CPB)], src_v)
        pltpu.sync_copy(dst_hbm.at[pl.ds(rbase, _CPB)], dst_v)
        pltpu.sync_copy(et_hbm.at[pl.ds(base, _BE)], et_v)

        pltpu.async_copy(h_hbm.at[src_v.at[0]], rows_a, sem_ga)

        def _pair(j, _1):
            c0 = 2 * j
            c1 = c0 + 1
            i0 = src_v.at[c0]
            i1 = src_v.at[c1]
            o0 = dst_v.at[c0]
            o1 = dst_v.at[c1]
            pltpu.async_copy(h_hbm.at[i1], rows_b, sem_gb)
            pltpu.make_async_copy(h_hbm.at[i0], rows_a, sem_ga).wait()

            @pl.when(j > 0)
            def _wa():
                pltpu.make_async_copy(msg_a, acc.at[o0], sem_sa).wait()

            _compute(c0, rows_a, msg_a)
            pltpu.async_copy(msg_a, acc.at[o0], sem_sa, add=True)

            @pl.when(j < _CPB // 2 - 1)
            def _pf():
                pltpu.async_copy(h_hbm.at[src_v.at[c0 + 2]], rows_a, sem_ga)

            pltpu.make_async_copy(h_hbm.at[i1], rows_b, sem_gb).wait()

            @pl.when(j > 0)
            def _wb():
                pltpu.make_async_copy(msg_b, acc.at[o1], sem_sb).wait()

            _compute(c1, rows_b, msg_b)
            pltpu.async_copy(msg_b, acc.at[o1], sem_sb, add=True)
            return 0

        lax.fori_loop(0, _CPB // 2, _pair, 0)

        pltpu.make_async_copy(msg_a, acc.at[dst_v.at[0]], sem_sa).wait()
        pltpu.make_async_copy(msg_b, acc.at[dst_v.at[0]], sem_sb).wait()
        return 0

    lax.fori_loop(0, _NBLK, _block, 0)

    plsc.subcore_barrier()

    pltpu.sync_copy(acc.at[pl.ds(s * _RPT, _RPT)],
                    out_hbm.at[c, pl.ds(s * _RPT, _RPT)])


def kernel(features, edge_index, edge_type, W1, b1, W2, b2,
           gamma1, beta1, gamma2, beta2):
    src = edge_index[0].reshape(_ROWS, _K)
    dst = edge_index[1].reshape(_ROWS, _K)
    et = _et_prep(edge_type)
    gb1 = jnp.stack([gamma1, beta1])
    gb2 = jnp.stack([gamma2, beta2])

    st1 = _stats(features)
    h = _bn_mm(st1, gb1, b1[None, :], features, W1)
    parts1 = _propagate(h, src, dst, et)
    h_new, st2 = _sum_stats(parts1)
    h2 = _bn_mm(st2, gb2, b2[None, :], h_new, W2)
    parts2 = _propagate(h2, src, dst, et)
    return _final(parts2, features)


# speedup vs baseline: 1.3238x; 1.3238x over previous
import functools

import jax
import jax.numpy as jnp
from jax import lax
from jax.experimental import pallas as pl
from jax.experimental.pallas import tpu as pltpu
from jax.experimental.pallas import tpu_sc as plsc

_N = 10000
_E = 320000
_D = 128
_T = 4
_DT = _D * _T

_NW = 32
_K = 25
_CPB = 40
_BE = _CPB * _K
_NBLK = 10
_EPW = _NBLK * _BE
_ROWS = _E // _K
_NPAD = 10112
_RPT = _NPAD // 16

_BLK = 1000

_ORDER = tuple(b + (j // 2) + 16 * (j % 2)
               for b in range(0, _DT, 32) for j in range(32))


def _stats_body(x_ref, o_ref):
    @pl.when(pl.program_id(0) == 0)
    def _init():
        o_ref[...] = jnp.zeros_like(o_ref)

    x = x_ref[...]
    o_ref[0:1, :] += jnp.sum(x, axis=0, keepdims=True)
    o_ref[1:2, :] += jnp.sum(x * x, axis=0, keepdims=True)


def _stats(x):
    return pl.pallas_call(
        _stats_body,
        grid=(_N // _BLK,),
        in_specs=[pl.BlockSpec((_BLK, _D), lambda i: (i, 0))],
        out_specs=pl.BlockSpec((2, _D), lambda i: (0, 0)),
        out_shape=jax.ShapeDtypeStruct((2, _D), jnp.float32),
    )(x)


def _sum_stats_body(p0_ref, p1_ref, h_ref, o_ref):
    @pl.when(pl.program_id(0) == 0)
    def _init():
        o_ref[...] = jnp.zeros_like(o_ref)

    h = p0_ref[0] + p1_ref[0]
    h_ref[...] = h
    o_ref[0:1, :] += jnp.sum(h, axis=0, keepdims=True)
    o_ref[1:2, :] += jnp.sum(h * h, axis=0, keepdims=True)


def _sum_stats(parts):
    return pl.pallas_call(
        _sum_stats_body,
        grid=(_N // _BLK,),
        in_specs=[
            pl.BlockSpec((1, _BLK, _D), lambda i: (0, i, 0)),
            pl.BlockSpec((1, _BLK, _D), lambda i: (1, i, 0)),
        ],
        out_specs=[
            pl.BlockSpec((_BLK, _D), lambda i: (i, 0)),
            pl.BlockSpec((2, _D), lambda i: (0, 0)),
        ],
        out_shape=[
            jax.ShapeDtypeStruct((_N, _D), jnp.float32),
            jax.ShapeDtypeStruct((2, _D), jnp.float32),
        ],
    )(parts, parts)


def _bn_mm_body(st_ref, gb_ref, b_ref, x_ref, w_ref, o_ref):
    inv_n = 1.0 / _N
    mean = st_ref[0:1, :] * inv_n
    var = st_ref[1:2, :] * inv_n - mean * mean
    scale = lax.rsqrt(var + 1e-5) * gb_ref[0:1, :]
    shift = gb_ref[1:2, :] - mean * scale
    xr = jnp.maximum(x_ref[...] * scale + shift, 0.0)
    h = (
        jnp.dot(xr, w_ref[...], preferred_element_type=jnp.float32) + b_ref[...]
    ).astype(jnp.bfloat16)
    for t in range(_T):
        o_ref[:, t, :] = h[:, t * _D:(t + 1) * _D]


def _bn_mm(st, gb, b, x, w):
    return pl.pallas_call(
        _bn_mm_body,
        grid=(_N // _BLK,),
        in_specs=[
            pl.BlockSpec((2, _D), lambda i: (0, 0)),
            pl.BlockSpec((2, _D), lambda i: (0, 0)),
            pl.BlockSpec((1, _DT), lambda i: (0, 0)),
            pl.BlockSpec((_BLK, _D), lambda i: (i, 0)),
            pl.BlockSpec((_D, _DT), lambda i: (0, 0)),
        ],
        out_specs=pl.BlockSpec((_BLK, _T, _D), lambda i: (i, 0, 0)),
        out_shape=jax.ShapeDtypeStruct((_N, _T, _D), jnp.bfloat16),
    )(st, gb, b, x, w)


def _final_body(p0_ref, p1_ref, f_ref, o_ref):
    o_ref[...] = p0_ref[0] + p1_ref[0] + f_ref[...]


def _final(parts, f):
    return pl.pallas_call(
        _final_body,
        grid=(_N // _BLK,),
        in_specs=[
            pl.BlockSpec((1, _BLK, _D), lambda i: (0, i, 0)),
            pl.BlockSpec((1, _BLK, _D), lambda i: (1, i, 0)),
            pl.BlockSpec((_BLK, _D), lambda i: (i, 0)),
        ],
        out_specs=pl.BlockSpec((_BLK, _D), lambda i: (i, 0)),
        out_shape=jax.ShapeDtypeStruct((_BLK * (_N // _BLK), _D), jnp.float32),
    )(parts, parts, f)


_mesh = plsc.VectorSubcoreMesh(core_axis_name="c", subcore_axis_name="s")


@functools.partial(
    pl.kernel,
    out_type=jax.ShapeDtypeStruct((2, _NPAD, _D), jnp.float32),
    mesh=_mesh,
    scratch_types=[
        pltpu.VMEM_SHARED((_NPAD, _D), jnp.float32),
        pltpu.VMEM((_CPB, _K), jnp.int32),
        pltpu.VMEM((_CPB, _K), jnp.int32),
        pltpu.VMEM((_CPB, _K * _T), jnp.float32),
        pltpu.VMEM((_K, _T, _D), jnp.bfloat16),
        pltpu.VMEM((_K, _T, _D), jnp.bfloat16),
        pltpu.VMEM((_K, _D), jnp.float32),
        pltpu.VMEM((_K, _D), jnp.float32),
        pltpu.SemaphoreType.DMA,
        pltpu.SemaphoreType.DMA,
        pltpu.SemaphoreType.DMA,
        pltpu.SemaphoreType.DMA,
    ],
    compiler_params=pltpu.CompilerParams(
        use_tc_tiling_on_sc=False, needs_layout_passes=False
    ),
)
def _propagate(h_hbm, ei_hbm, et_hbm, out_hbm,
               acc, src_v, dst_v, et_v, rows_a, rows_b, msg_a, msg_b,
               sem_ga, sem_gb, sem_sa, sem_sb):
    c = lax.axis_index("c")
    s = lax.axis_index("s")
    wid = s * 2 + c

    def _zb(i, _):
        msg_a[i // 8, pl.ds((i % 8) * 16, 16)] = jnp.zeros((16,), jnp.float32)
        return 0

    lax.fori_loop(0, _K * 8, _zb, 0)

    def _zc(j, _):
        pltpu.sync_copy(msg_a, acc.at[pl.ds(s * _RPT + j * _K, _K)])
        return 0

    lax.fori_loop(0, _RPT // _K, _zc, 0)
    pltpu.sync_copy(msg_a.at[pl.ds(0, _RPT % _K)],
                    acc.at[pl.ds(s * _RPT + (_RPT // _K) * _K, _RPT % _K)])

    plsc.subcore_barrier()

    def _compute(ci, rows, msg):
        row_i = jnp.full((16,), ci, jnp.int32)

        @plsc.parallel_loop(0, _K, 1, unroll=2)
        def _edge(k):
            kb = k * _T
            w0 = plsc.load_gather(
                et_v, [row_i, jnp.full((16,), kb, jnp.int32)])
            w1 = plsc.load_gather(
                et_v, [row_i, jnp.full((16,), kb + 1, jnp.int32)])
            w2 = plsc.load_gather(
                et_v, [row_i, jnp.full((16,), kb + 2, jnp.int32)])
            w3 = plsc.load_gather(
                et_v, [row_i, jnp.full((16,), kb + 3, jnp.int32)])
            ws = (w0, w1, w2, w3)
            for g in range(_D // 32):
                o = g * 32
                m_lo = None
                m_hi = None
                for t in range(_T):
                    lo, hi = plsc.unpack(
                        rows[k, t, pl.ds(o, 32)],
                        format=plsc.PackFormat.INTERLEAVED,
                        preferred_element_type=jnp.float32)
                    if t == 0:
                        m_lo = ws[t] * lo
                        m_hi = ws[t] * hi
                    else:
                        m_lo = m_lo + ws[t] * lo
                        m_hi = m_hi + ws[t] * hi
                msg[k, pl.ds(o, 16)] = m_lo
                msg[k, pl.ds(o + 16, 16)] = m_hi

    def _block(b, _):
        base = wid * (_EPW // _K) + b * _CPB

        pltpu.sync_copy(ei_hbm.at[0, pl.ds(base, _CPB)], src_v)
        pltpu.sync_copy(ei_hbm.at[1, pl.ds(base, _CPB)], dst_v)
        pltpu.sync_copy(et_hbm.at[pl.ds(base, _CPB)], et_v)

        pltpu.async_copy(h_hbm.at[src_v.at[0]], rows_a, sem_ga)

        def _pair(j, _1):
            c0 = 2 * j
            c1 = c0 + 1
            pltpu.async_copy(h_hbm.at[src_v.at[c1]], rows_b, sem_gb)
            pltpu.make_async_copy(
                h_hbm.at[src_v.at[c0]], rows_a, sem_ga).wait()

            @pl.when(j > 0)
            def _wa():
                pltpu.make_async_copy(
                    msg_a, acc.at[dst_v.at[c0]], sem_sa).wait()

            _compute(c0, rows_a, msg_a)
            pltpu.async_copy(msg_a, acc.at[dst_v.at[c0]], sem_sa, add=True)

            @pl.when(j < _CPB // 2 - 1)
            def _pf():
                pltpu.async_copy(h_hbm.at[src_v.at[c0 + 2]], rows_a, sem_ga)

            pltpu.make_async_copy(
                h_hbm.at[src_v.at[c1]], rows_b, sem_gb).wait()

            @pl.when(j > 0)
            def _wb():
                pltpu.make_async_copy(
                    msg_b, acc.at[dst_v.at[c1]], sem_sb).wait()

            _compute(c1, rows_b, msg_b)
            pltpu.async_copy(msg_b, acc.at[dst_v.at[c1]], sem_sb, add=True)
            return 0

        lax.fori_loop(0, _CPB // 2, _pair, 0)

        pltpu.make_async_copy(msg_a, acc.at[dst_v.at[0]], sem_sa).wait()
        pltpu.make_async_copy(msg_b, acc.at[dst_v.at[0]], sem_sb).wait()
        return 0

    lax.fori_loop(0, _NBLK, _block, 0)

    plsc.subcore_barrier()

    pltpu.sync_copy(acc.at[pl.ds(s * _RPT, _RPT)],
                    out_hbm.at[c, pl.ds(s * _RPT, _RPT)])


def kernel(features, edge_index, edge_type, W1, b1, W2, b2,
           gamma1, beta1, gamma2, beta2):
    ei = edge_index.reshape(2, _ROWS, _K)
    et = edge_type.reshape(_ROWS, _K * _T)
    gb1 = jnp.stack([gamma1, beta1])
    gb2 = jnp.stack([gamma2, beta2])
    order = jnp.array(_ORDER, dtype=jnp.int32)
    w1p = W1[:, order]
    b1p = b1[order]
    w2p = W2[:, order]
    b2p = b2[order]

    st1 = _stats(features)
    h = _bn_mm(st1, gb1, b1p[None, :], features, w1p)
    parts1 = _propagate(h, ei, et)
    h_new, st2 = _sum_stats(parts1)
    h2 = _bn_mm(st2, gb2, b2p[None, :], h_new, w2p)
    parts2 = _propagate(h2, ei, et)
    return _final(parts2, features)


# speedup vs baseline: 1.5150x; 1.1444x over previous
import functools

import jax
import jax.numpy as jnp
from jax import lax
from jax.experimental import pallas as pl
from jax.experimental.pallas import tpu as pltpu
from jax.experimental.pallas import tpu_sc as plsc

_N = 10000
_E = 320000
_D = 128
_T = 4
_DT = _D * _T

_NW = 32
_K = 50
_CPB = 20
_BE = _CPB * _K
_NBLK = 10
_EPW = _NBLK * _BE
_ROWS = _E // _K
_NPAD = 10112
_RPT = _NPAD // 16

_BLK = 1000

_ORDER = tuple(b + (j // 2) + 16 * (j % 2)
               for b in range(0, _DT, 32) for j in range(32))


def _stats_body(x_ref, o_ref):
    @pl.when(pl.program_id(0) == 0)
    def _init():
        o_ref[...] = jnp.zeros_like(o_ref)

    x = x_ref[...]
    o_ref[0:1, :] += jnp.sum(x, axis=0, keepdims=True)
    o_ref[1:2, :] += jnp.sum(x * x, axis=0, keepdims=True)


def _stats(x):
    return pl.pallas_call(
        _stats_body,
        grid=(_N // _BLK,),
        in_specs=[pl.BlockSpec((_BLK, _D), lambda i: (i, 0))],
        out_specs=pl.BlockSpec((2, _D), lambda i: (0, 0)),
        out_shape=jax.ShapeDtypeStruct((2, _D), jnp.float32),
    )(x)


def _sum_stats_body(p0_ref, p1_ref, h_ref, o_ref):
    @pl.when(pl.program_id(0) == 0)
    def _init():
        o_ref[...] = jnp.zeros_like(o_ref)

    h = p0_ref[0] + p1_ref[0]
    h_ref[...] = h
    o_ref[0:1, :] += jnp.sum(h, axis=0, keepdims=True)
    o_ref[1:2, :] += jnp.sum(h * h, axis=0, keepdims=True)


def _sum_stats(parts):
    return pl.pallas_call(
        _sum_stats_body,
        grid=(_N // _BLK,),
        in_specs=[
            pl.BlockSpec((1, _BLK, _D), lambda i: (0, i, 0)),
            pl.BlockSpec((1, _BLK, _D), lambda i: (1, i, 0)),
        ],
        out_specs=[
            pl.BlockSpec((_BLK, _D), lambda i: (i, 0)),
            pl.BlockSpec((2, _D), lambda i: (0, 0)),
        ],
        out_shape=[
            jax.ShapeDtypeStruct((_N, _D), jnp.float32),
            jax.ShapeDtypeStruct((2, _D), jnp.float32),
        ],
    )(parts, parts)


def _bn_mm_body(st_ref, gb_ref, b_ref, x_ref, w_ref, o_ref):
    inv_n = 1.0 / _N
    mean = st_ref[0:1, :] * inv_n
    var = st_ref[1:2, :] * inv_n - mean * mean
    scale = lax.rsqrt(var + 1e-5) * gb_ref[0:1, :]
    shift = gb_ref[1:2, :] - mean * scale
    xr = jnp.maximum(x_ref[...] * scale + shift, 0.0)
    h = (
        jnp.dot(xr, w_ref[...], preferred_element_type=jnp.float32) + b_ref[...]
    ).astype(jnp.bfloat16)
    o_ref[...] = h.reshape(_BLK, _T, _D)


def _bn_mm(st, gb, b, x, w):
    return pl.pallas_call(
        _bn_mm_body,
        grid=(_N // _BLK,),
        in_specs=[
            pl.BlockSpec((2, _D), lambda i: (0, 0)),
            pl.BlockSpec((2, _D), lambda i: (0, 0)),
            pl.BlockSpec((1, _DT), lambda i: (0, 0)),
            pl.BlockSpec((_BLK, _D), lambda i: (i, 0)),
            pl.BlockSpec((_D, _DT), lambda i: (0, 0)),
        ],
        out_specs=pl.BlockSpec((_BLK, _T, _D), lambda i: (i, 0, 0)),
        out_shape=jax.ShapeDtypeStruct((_N, _T, _D), jnp.bfloat16),
    )(st, gb, b, x, w)


def _final_body(p0_ref, p1_ref, f_ref, o_ref):
    o_ref[...] = p0_ref[0] + p1_ref[0] + f_ref[...]


def _final(parts, f):
    return pl.pallas_call(
        _final_body,
        grid=(_N // _BLK,),
        in_specs=[
            pl.BlockSpec((1, _BLK, _D), lambda i: (0, i, 0)),
            pl.BlockSpec((1, _BLK, _D), lambda i: (1, i, 0)),
            pl.BlockSpec((_BLK, _D), lambda i: (i, 0)),
        ],
        out_specs=pl.BlockSpec((_BLK, _D), lambda i: (i, 0)),
        out_shape=jax.ShapeDtypeStruct((_BLK * (_N // _BLK), _D), jnp.float32),
    )(parts, parts, f)


_mesh = plsc.VectorSubcoreMesh(core_axis_name="c", subcore_axis_name="s")


@functools.partial(
    pl.kernel,
    out_type=jax.ShapeDtypeStruct((2, _NPAD, _D), jnp.float32),
    mesh=_mesh,
    scratch_types=[
        pltpu.VMEM_SHARED((_NPAD, _D), jnp.float32),
        pltpu.VMEM((_CPB, _K), jnp.int32),
        pltpu.VMEM((_CPB, _K), jnp.int32),
        pltpu.VMEM((_CPB, _K * _T), jnp.float32),
        pltpu.VMEM((_K, _T, _D), jnp.bfloat16),
        pltpu.VMEM((_K, _T, _D), jnp.bfloat16),
        pltpu.VMEM((_K, _D), jnp.float32),
        pltpu.VMEM((_K, _D), jnp.float32),
        pltpu.SemaphoreType.DMA,
        pltpu.SemaphoreType.DMA,
        pltpu.SemaphoreType.DMA,
        pltpu.SemaphoreType.DMA,
    ],
    compiler_params=pltpu.CompilerParams(
        use_tc_tiling_on_sc=False, needs_layout_passes=False
    ),
)
def _propagate(h_hbm, ei_hbm, et_hbm, out_hbm,
               acc, src_v, dst_v, et_v, rows_a, rows_b, msg_a, msg_b,
               sem_ga, sem_gb, sem_sa, sem_sb):
    c = lax.axis_index("c")
    s = lax.axis_index("s")
    wid = s * 2 + c

    def _zb(i, _):
        msg_a[i // 8, pl.ds((i % 8) * 16, 16)] = jnp.zeros((16,), jnp.float32)
        return 0

    lax.fori_loop(0, _K * 8, _zb, 0)

    def _zc(j, _):
        pltpu.sync_copy(msg_a, acc.at[pl.ds(s * _RPT + j * _K, _K)])
        return 0

    lax.fori_loop(0, _RPT // _K, _zc, 0)
    pltpu.sync_copy(msg_a.at[pl.ds(0, _RPT % _K)],
                    acc.at[pl.ds(s * _RPT + (_RPT // _K) * _K, _RPT % _K)])

    plsc.subcore_barrier()

    def _compute(ci, rows, msg):
        row_i = jnp.full((16,), ci, jnp.int32)

        @plsc.parallel_loop(0, _K, 1, unroll=2)
        def _edge(k):
            kb = k * _T
            w0 = plsc.load_gather(
                et_v, [row_i, jnp.full((16,), kb, jnp.int32)])
            w1 = plsc.load_gather(
                et_v, [row_i, jnp.full((16,), kb + 1, jnp.int32)])
            w2 = plsc.load_gather(
                et_v, [row_i, jnp.full((16,), kb + 2, jnp.int32)])
            w3 = plsc.load_gather(
                et_v, [row_i, jnp.full((16,), kb + 3, jnp.int32)])
            ws = (w0, w1, w2, w3)
            for g in range(_D // 32):
                o = g * 32
                m_lo = None
                m_hi = None
                for t in range(_T):
                    lo, hi = plsc.unpack(
                        rows[k, t, pl.ds(o, 32)],
                        format=plsc.PackFormat.INTERLEAVED,
                        preferred_element_type=jnp.float32)
                    if t == 0:
                        m_lo = ws[t] * lo
                        m_hi = ws[t] * hi
                    else:
                        m_lo = m_lo + ws[t] * lo
                        m_hi = m_hi + ws[t] * hi
                msg[k, pl.ds(o, 16)] = m_lo
                msg[k, pl.ds(o + 16, 16)] = m_hi

    def _block(b, _):
        base = wid * (_EPW // _K) + b * _CPB

        pltpu.sync_copy(ei_hbm.at[0, pl.ds(base, _CPB)], src_v)
        pltpu.sync_copy(ei_hbm.at[1, pl.ds(base, _CPB)], dst_v)
        pltpu.sync_copy(et_hbm.at[pl.ds(base, _CPB)], et_v)

        pltpu.async_copy(h_hbm.at[src_v.at[0]], rows_a, sem_ga)

        def _pair(j, _1):
            c0 = 2 * j
            c1 = c0 + 1
            pltpu.async_copy(h_hbm.at[src_v.at[c1]], rows_b, sem_gb)
            pltpu.make_async_copy(
                h_hbm.at[src_v.at[c0]], rows_a, sem_ga).wait()

            @pl.when(j > 0)
            def _wa():
                pltpu.make_async_copy(
                    msg_a, acc.at[dst_v.at[c0]], sem_sa).wait()

            _compute(c0, rows_a, msg_a)
            pltpu.async_copy(msg_a, acc.at[dst_v.at[c0]], sem_sa, add=True)

            @pl.when(j < _CPB // 2 - 1)
            def _pf():
                pltpu.async_copy(h_hbm.at[src_v.at[c0 + 2]], rows_a, sem_ga)

            pltpu.make_async_copy(
                h_hbm.at[src_v.at[c1]], rows_b, sem_gb).wait()

            @pl.when(j > 0)
            def _wb():
                pltpu.make_async_copy(
                    msg_b, acc.at[dst_v.at[c1]], sem_sb).wait()

            _compute(c1, rows_b, msg_b)
            pltpu.async_copy(msg_b, acc.at[dst_v.at[c1]], sem_sb, add=True)
            return 0

        lax.fori_loop(0, _CPB // 2, _pair, 0)

        pltpu.make_async_copy(msg_a, acc.at[dst_v.at[0]], sem_sa).wait()
        pltpu.make_async_copy(msg_b, acc.at[dst_v.at[0]], sem_sb).wait()
        return 0

    lax.fori_loop(0, _NBLK, _block, 0)

    plsc.subcore_barrier()

    pltpu.sync_copy(acc.at[pl.ds(s * _RPT, _RPT)],
                    out_hbm.at[c, pl.ds(s * _RPT, _RPT)])


def kernel(features, edge_index, edge_type, W1, b1, W2, b2,
           gamma1, beta1, gamma2, beta2):
    ei = edge_index.reshape(2, _ROWS, _K)
    et = edge_type.reshape(_ROWS, _K * _T)
    gb1 = jnp.stack([gamma1, beta1])
    gb2 = jnp.stack([gamma2, beta2])
    order = jnp.array(_ORDER, dtype=jnp.int32)
    w1p = W1[:, order]
    b1p = b1[order]
    w2p = W2[:, order]
    b2p = b2[order]

    st1 = _stats(features)
    h = _bn_mm(st1, gb1, b1p[None, :], features, w1p)
    parts1 = _propagate(h, ei, et)
    h_new, st2 = _sum_stats(parts1)
    h2 = _bn_mm(st2, gb2, b2p[None, :], h_new, w2p)
    parts2 = _propagate(h2, ei, et)
    return _final(parts2, features)
